# baseline (speedup 1.0000x reference)
import functools

import jax
import jax.numpy as jnp
from jax import lax
from jax.experimental import pallas as pl
from jax.experimental.pallas import tpu as pltpu
from jax.experimental.pallas import tpu_sc as plsc

_info = plsc.get_sparse_core_info()
_NC, _NS, _L = _info.num_cores, _info.num_subcores, _info.num_lanes
_NW = _NC * _NS


def _make_sc_kernel(V, D, E):
    assert E % _NW == 0
    e_w = E // _NW
    B = 400
    assert e_w % B == 0 and B % _L == 0 and D % (2 * _L) == 0
    n_chunks = e_w // B
    assert n_chunks % 2 == 1
    n_pairs = (n_chunks - 1) // 2
    groups = B // _L
    Dw = D // 2
    n_sl = Dw // _L

    mesh = plsc.VectorSubcoreMesh(core_axis_name="c", subcore_axis_name="s")

    @functools.partial(
        pl.kernel,
        mesh=mesh,
        compiler_params=pltpu.CompilerParams(
            needs_layout_passes=False, use_tc_tiling_on_sc=False),
        out_type=jax.ShapeDtypeStruct((E,), jnp.float32),
        scratch_types=[
            pltpu.VMEM((B, Dw), jnp.int32),
            pltpu.VMEM((B, Dw), jnp.int32),
            pltpu.VMEM((B, Dw), jnp.int32),
            pltpu.VMEM((B, Dw), jnp.int32),
            pltpu.VMEM((B,), jnp.int32),
            pltpu.VMEM((B,), jnp.int32),
            pltpu.VMEM((B,), jnp.int32),
            pltpu.VMEM((B,), jnp.int32),
            pltpu.VMEM((D,), jnp.float32),
            pltpu.VMEM((e_w,), jnp.float32),
            pltpu.SemaphoreType.DMA,
            pltpu.SemaphoreType.DMA,
            pltpu.SemaphoreType.DMA,
            pltpu.SemaphoreType.DMA,
            pltpu.SemaphoreType.DMA,
            pltpu.SemaphoreType.DMA,
        ],
    )
    def k(table_hbm, ridx_hbm, cidx_hbm, w_hbm, out_hbm,
          rbuf_a, cbuf_a, rbuf_b, cbuf_b,
          ridx_a, cidx_a, ridx_b, cidx_b, w_v, out_v,
          sem_ra, sem_ca, sem_rb, sem_cb, sem_ia, sem_ib):
        wid = lax.axis_index("s") * _NC + lax.axis_index("c")
        base = wid * e_w
        pltpu.sync_copy(w_hbm, w_v)

        lanes = lax.iota(jnp.int32, _L)
        zero = jnp.zeros((_L,), jnp.float32)

        def idx_copy(ch, ridx_s, cidx_s, sem_i):
            off = base + ch * B
            pltpu.async_copy(ridx_hbm.at[pl.ds(off, B)], ridx_s, sem_i)
            pltpu.async_copy(cidx_hbm.at[pl.ds(off, B)], cidx_s, sem_i)

        def idx_wait(ridx_s, cidx_s, sem_i):
            pltpu.make_async_copy(
                ridx_hbm.at[pl.ds(0, B)], ridx_s, sem_i).wait()
            pltpu.make_async_copy(
                cidx_hbm.at[pl.ds(0, B)], cidx_s, sem_i).wait()

        def issue(ridx_s, cidx_s, rbuf, cbuf, sem_r, sem_c):
            pltpu.async_copy(table_hbm.at[ridx_s], rbuf, sem_r)
            pltpu.async_copy(table_hbm.at[cidx_s], cbuf, sem_c)

        def wait(ridx_s, cidx_s, rbuf, cbuf, sem_r, sem_c):
            pltpu.make_async_copy(
                table_hbm.at[ridx_s], rbuf, sem_r).wait()
            pltpu.make_async_copy(
                table_hbm.at[cidx_s], cbuf, sem_c).wait()

        we_regs = [w_v[pl.ds(i * _L, _L)] for i in range(n_sl)]
        wo_regs = [w_v[pl.ds(Dw + i * _L, _L)] for i in range(n_sl)]

        def compute(ch, rbuf, cbuf):
            off = ch * B

            def group_body(eb, _):
                def quad_body(q, res):
                    for k in range(4):
                        j = q * 4 + k
                        e = eb * _L + j
                        acc_e = zero
                        acc_o = zero
                        for i in range(n_sl):
                            rv = plsc.bitcast(
                                rbuf[e, pl.ds(i * _L, _L)], jnp.bfloat16)
                            cv = plsc.bitcast(
                                cbuf[e, pl.ds(i * _L, _L)], jnp.bfloat16)
                            da, db = plsc.unpack(
                                jnp.abs(rv - cv),
                                format=plsc.PackFormat.INTERLEAVED)
                            acc_e = acc_e + da * we_regs[i]
                            acc_o = acc_o + db * wo_regs[i]
                        s = jnp.sum(acc_e + acc_o)
                        res = jnp.where(lanes == j, s, res)
                    return res

                res = lax.fori_loop(0, 4, quad_body, zero)
                sig = 1.0 / (1.0 + jnp.exp(-res))
                out_v[pl.ds(off + eb * _L, _L)] = sig
                return 0

            lax.fori_loop(0, groups, group_body, 0)

        idx_copy(0, ridx_a, cidx_a, sem_ia)
        idx_wait(ridx_a, cidx_a, sem_ia)
        issue(ridx_a, cidx_a, rbuf_a, cbuf_a, sem_ra, sem_ca)
        idx_copy(1, ridx_b, cidx_b, sem_ib)

        def pair_body(p, _):
            ch = 2 * p
            idx_wait(ridx_b, cidx_b, sem_ib)
            issue(ridx_b, cidx_b, rbuf_b, cbuf_b, sem_rb, sem_cb)
            wait(ridx_a, cidx_a, rbuf_a, cbuf_a, sem_ra, sem_ca)
            idx_copy(ch + 2, ridx_a, cidx_a, sem_ia)
            compute(ch, rbuf_a, cbuf_a)
            idx_wait(ridx_a, cidx_a, sem_ia)
            issue(ridx_a, cidx_a, rbuf_a, cbuf_a, sem_ra, sem_ca)
            wait(ridx_b, cidx_b, rbuf_b, cbuf_b, sem_rb, sem_cb)

            @pl.when(p + 1 < n_pairs)
            def _():
                idx_copy(ch + 3, ridx_b, cidx_b, sem_ib)

            compute(ch + 1, rbuf_b, cbuf_b)
            return 0

        lax.fori_loop(0, n_pairs, pair_body, 0)
        wait(ridx_a, cidx_a, rbuf_a, cbuf_a, sem_ra, sem_ca)
        compute(n_chunks - 1, rbuf_a, cbuf_a)

        pltpu.sync_copy(out_v, out_hbm.at[pl.ds(base, e_w)])

    return k


def kernel(inputs, r_indices, c_indices, weights):
    V, D = inputs.shape
    E = r_indices.shape[0]
    r32 = r_indices.astype(jnp.int32)
    c32 = c_indices.astype(jnp.int32)
    t16 = inputs.astype(jnp.bfloat16).reshape(V, D // 2, 2)
    t32 = jax.lax.bitcast_convert_type(t16, jnp.int32)
    w = weights.reshape(-1).astype(jnp.float32)
    w_de = jnp.concatenate([w[0::2], w[1::2]])
    k = _make_sc_kernel(V, D, E)
    return k(t32, r32, c32, w_de)
